# baseline (speedup 1.0000x reference)
import functools

import jax
import jax.numpy as jnp
from jax import lax
from jax.experimental import pallas as pl
from jax.experimental.pallas import tpu as pltpu
from jax.experimental.pallas import tpu_sc as plsc

N = 10000
E = 320000
G = 32
NC = 2
NS = 16
RB = 1000
NB = N // RB
CH = 80
PER_W = E // (NC * NS)
ITERS = 128
KP = 4
NG = ITERS // KP
NSINK = 8
RS_SMALL = 624
RS_BIG = N - RS_SMALL * (NS - 1)


def _row_split(s, fn):
    @pl.when(s < NS - 1)
    def _():
        fn(s * RS_SMALL, RS_SMALL)

    @pl.when(s == NS - 1)
    def _():
        fn(RS_SMALL * (NS - 1), RS_BIG)

_f32 = jnp.float32


def _sc_mesh():
    return plsc.VectorSubcoreMesh(
        core_axis_name="c", subcore_axis_name="s",
        num_cores=NC, num_subcores=NS)


@functools.lru_cache(maxsize=None)
def _make_degree():
    @functools.partial(
        pl.kernel,
        out_type=jax.ShapeDtypeStruct((NC, N, 128), _f32),
        mesh=_sc_mesh(),
        scratch_types=[
            pltpu.VMEM((NG, KP, CH), jnp.int32),
            pltpu.VMEM((CH, 128), _f32),
            pltpu.VMEM_SHARED((N + NSINK, 128), _f32),
            pltpu.SemaphoreType.DMA,
        ],
    )
    def deg_kernel(dstr_hbm, ones_hbm, zeros_hbm, out_hbm,
                   dst_all, ones_v, acc_sh, ssem):
        c = lax.axis_index("c")
        s = lax.axis_index("s")
        wid = c * NS + s
        pltpu.sync_copy(ones_hbm, ones_v)
        pltpu.sync_copy(dstr_hbm.at[wid], dst_all)
        _row_split(s, lambda r0, nr: pltpu.sync_copy(
            zeros_hbm.at[pl.ds(0, nr)], acc_sh.at[pl.ds(r0, nr)]))
        plsc.subcore_barrier()

        def body(g, carry):
            for t in range(KP):
                pltpu.async_copy(ones_v, acc_sh.at[dst_all.at[g, t]],
                                 ssem, add=True)
            return carry

        lax.fori_loop(0, NG, body, 0)

        def drain(i, carry):
            pltpu.make_async_copy(
                zeros_hbm.at[pl.ds(0, CH)], ones_v, ssem).wait()
            return carry

        lax.fori_loop(0, ITERS, drain, 0)
        plsc.subcore_barrier()
        _row_split(s, lambda r0, nr: pltpu.sync_copy(
            acc_sh.at[pl.ds(r0, nr)], out_hbm.at[c, pl.ds(r0, nr)]))

    return deg_kernel


@functools.lru_cache(maxsize=None)
def _make_prop(num_blocks):
    scratch = (
        [pltpu.VMEM((CH,), jnp.int32) for _ in range(2 * KP)]
        + [pltpu.VMEM((CH,), jnp.int32) for _ in range(2 * KP)]
        + [
            pltpu.VMEM((KP, CH, 128), _f32),
            pltpu.VMEM_SHARED((N + NSINK, 128), _f32),
            pltpu.SemaphoreType.DMA,
            pltpu.SemaphoreType.DMA,
            pltpu.SemaphoreType.DMA,
        ])
    out_type = [jax.ShapeDtypeStruct((NC, N, 128), _f32)
                for _ in range(num_blocks)]

    @functools.partial(pl.kernel, out_type=out_type, mesh=_sc_mesh(),
                       scratch_types=scratch)
    def prop_kernel(srcr_hbm, dstr_hbm, zeros_hbm, *rest):
        pieces = rest[:num_blocks]
        outs = rest[num_blocks:2 * num_blocks]
        rest = rest[2 * num_blocks:]
        srcb = rest[:2 * KP]
        dstb = rest[2 * KP:4 * KP]
        rows, acc_sh, gsem, ssem, isem = rest[4 * KP:]
        c = lax.axis_index("c")
        s = lax.axis_index("s")
        wid = c * NS + s

        def idx_load(j, half):
            for t in range(KP):
                row = (wid * NG + j) * KP + t
                pltpu.async_copy(srcr_hbm.at[row], srcb[half * KP + t], isem)
                pltpu.async_copy(dstr_hbm.at[row], dstb[half * KP + t], isem)

        def idx_wait():
            for _t in range(2 * KP):
                pltpu.make_async_copy(
                    srcr_hbm.at[0], srcb[0], isem).wait()

        def drain_scatter():
            pltpu.make_async_copy(
                zeros_hbm.at[pl.ds(0, CH)], rows.at[0], ssem).wait()

        for b in range(num_blocks):
            _row_split(s, lambda r0, nr: pltpu.sync_copy(
                zeros_hbm.at[pl.ds(0, nr)], acc_sh.at[pl.ds(r0, nr)]))
            idx_load(0, 0)
            plsc.subcore_barrier()

            def group(j, p, b):
                idx_wait()

                @pl.when(j >= 1)
                def _():
                    for _t in range(KP):
                        drain_scatter()

                @pl.when(j + 1 < NG)
                def _():
                    idx_load(j + 1, 1 - p)

                descs = []
                for t in range(KP):
                    descs.append(pltpu.async_copy(
                        pieces[b].at[srcb[p * KP + t]],
                        rows.at[t], gsem))
                for t in range(KP):
                    descs[t].wait()
                for t in range(KP):
                    pltpu.async_copy(rows.at[t],
                                     acc_sh.at[dstb[p * KP + t]],
                                     ssem, add=True)

            def body(k, carry, b=b):
                group(2 * k, 0, b)
                group(2 * k + 1, 1, b)
                return carry

            lax.fori_loop(0, NG // 2, body, 0)
            for _t in range(KP):
                drain_scatter()
            plsc.subcore_barrier()
            _row_split(s, lambda r0, nr, b=b: pltpu.sync_copy(
                acc_sh.at[pl.ds(r0, nr)], outs[b].at[c, pl.ds(r0, nr)]))
            plsc.subcore_barrier()

    return prop_kernel


def _sc_degree(dst, ones128, zeros128):
    dstr = _pad_idx(dst, N)
    return _make_degree()(dstr, ones128, zeros128)


def _pad_idx(idx, fill):
    per_w = idx.reshape(NC * NS, PER_W)
    padded = jnp.pad(per_w, ((0, 0), (0, ITERS * CH - PER_W)),
                     constant_values=fill)
    return padded.reshape(NC * NS, NG, KP, CH)


def _sc_prop(src, dst, zeros128, pieces):
    srcr = _pad_idx(src, 0).reshape(-1, CH)
    dstr = _pad_idx(dst, N).reshape(-1, CH)
    outs = _make_prop(len(pieces))(srcr, dstr, zeros128, *pieces)
    return list(outs)


def _finalize_deg(degparts, x):
    def body(dp_ref, x_ref, dinv_ref, y0_ref):
        deg = dp_ref[0, :, 0:1] + dp_ref[1, :, 0:1] + 1.0
        dinv = lax.rsqrt(deg)
        dinv_ref[...] = jnp.broadcast_to(dinv, (RB, 8))
        y0_ref[...] = x_ref[...] * dinv

    return pl.pallas_call(
        body,
        grid=(NB,),
        in_specs=[
            pl.BlockSpec((NC, RB, 128), lambda b: (0, b, 0)),
            pl.BlockSpec((RB, 128), lambda b: (b, 0)),
        ],
        out_specs=[
            pl.BlockSpec((RB, 8), lambda b: (b, 0)),
            pl.BlockSpec((RB, 128), lambda b: (b, 0)),
        ],
        out_shape=[
            jax.ShapeDtypeStruct((N, 8), _f32),
            jax.ShapeDtypeStruct((N, 128), _f32),
        ],
    )(degparts, x)


def _combine(parts, y, dinv8):
    def body(p_ref, y_ref, dinv_ref, o_ref):
        o_ref[...] = ((p_ref[0] + p_ref[1] + y_ref[...])
                      * dinv_ref[:, 0:1])

    return pl.pallas_call(
        body,
        grid=(NB,),
        in_specs=[
            pl.BlockSpec((NC, RB, 128), lambda b: (0, b, 0)),
            pl.BlockSpec((RB, 128), lambda b: (b, 0)),
            pl.BlockSpec((RB, 8), lambda b: (b, 0)),
        ],
        out_specs=pl.BlockSpec((RB, 128), lambda b: (b, 0)),
        out_shape=jax.ShapeDtypeStruct((N, 128), _f32),
    )(parts, y, dinv8)


def _matmul_stats(pieces, w):
    npc = len(pieces)
    wo = w.shape[1]

    def body(*refs):
        piece_refs = refs[:npc]
        w_ref = refs[npc]
        t_ref, st_ref = refs[npc + 1:]
        b = pl.program_id(0)
        acc = jnp.zeros((RB, wo), _f32)
        for j in range(npc):
            acc = acc + jnp.dot(piece_refs[j][...],
                                w_ref[j * 128:(j + 1) * 128, :],
                                preferred_element_type=_f32,
                                precision=lax.Precision.HIGHEST)
        t_ref[...] = acc

        @pl.when(b == 0)
        def _():
            st_ref[...] = jnp.zeros((8, wo), _f32)

        st_ref[0:1, :] = st_ref[0:1, :] + jnp.sum(acc, 0, keepdims=True)
        st_ref[1:2, :] = st_ref[1:2, :] + jnp.sum(acc * acc, 0, keepdims=True)

    return pl.pallas_call(
        body,
        grid=(NB,),
        in_specs=[pl.BlockSpec((RB, 128), lambda b: (b, 0))
                  for _ in range(npc)]
                 + [pl.BlockSpec((128 * npc, wo), lambda b: (0, 0))],
        out_specs=[
            pl.BlockSpec((RB, wo), lambda b: (b, 0)),
            pl.BlockSpec((8, wo), lambda b: (0, 0)),
        ],
        out_shape=[
            jax.ShapeDtypeStruct((N, wo), _f32),
            jax.ShapeDtypeStruct((8, wo), _f32),
        ],
    )(*pieces, w)


def _bn_apply(t, stats, gb, dinv8, batchf, nys):
    wo = t.shape[1]

    def body(*refs):
        t_ref, st_ref, gb_ref, dinv_ref, bf_ref, pool_ref = refs[:6]
        ys_refs = refs[6:]
        b = pl.program_id(0)
        m = st_ref[0:1, :] * (1.0 / N)
        var = st_ref[1:2, :] * (1.0 / N) - m * m
        t_blk = t_ref[...]
        h = gb_ref[0:1, :] * (t_blk - m) * lax.rsqrt(var + 1e-5) + gb_ref[1:2, :]
        h = jnp.maximum(h, 0.0)
        for j in range(nys):
            ys_refs[j][...] = h[:, j * 128:(j + 1) * 128] * dinv_ref[:, 0:1]

        @pl.when(b == 0)
        def _():
            pool_ref[...] = jnp.full((G, wo), -jnp.inf, _f32)

        bf = bf_ref[...]
        for g in range(G):
            hm = jnp.where(bf == float(g), h, -jnp.inf)
            mx = jnp.max(hm, axis=0, keepdims=True)
            pool_ref[g:g + 1, :] = jnp.maximum(pool_ref[g:g + 1, :], mx)

    outs = pl.pallas_call(
        body,
        grid=(NB,),
        in_specs=[
            pl.BlockSpec((RB, wo), lambda b: (b, 0)),
            pl.BlockSpec((8, wo), lambda b: (0, 0)),
            pl.BlockSpec((8, wo), lambda b: (0, 0)),
            pl.BlockSpec((RB, 8), lambda b: (b, 0)),
            pl.BlockSpec((RB, 1), lambda b: (b, 0)),
        ],
        out_specs=[pl.BlockSpec((G, wo), lambda b: (0, 0))]
                  + [pl.BlockSpec((RB, 128), lambda b: (b, 0))
                     for _ in range(nys)],
        out_shape=[jax.ShapeDtypeStruct((G, wo), _f32)]
                  + [jax.ShapeDtypeStruct((N, 128), _f32)
                     for _ in range(nys)],
    )(t, stats, gb, dinv8, batchf)
    return outs[0], list(outs[1:])


def _segmax(x, batchf):
    wo = x.shape[1]

    def body(x_ref, bf_ref, pool_ref):
        b = pl.program_id(0)

        @pl.when(b == 0)
        def _():
            pool_ref[...] = jnp.full((G, wo), -jnp.inf, _f32)

        h = x_ref[...]
        bf = bf_ref[...]
        for g in range(G):
            hm = jnp.where(bf == float(g), h, -jnp.inf)
            mx = jnp.max(hm, axis=0, keepdims=True)
            pool_ref[g:g + 1, :] = jnp.maximum(pool_ref[g:g + 1, :], mx)

    return pl.pallas_call(
        body,
        grid=(NB,),
        in_specs=[
            pl.BlockSpec((RB, wo), lambda b: (b, 0)),
            pl.BlockSpec((RB, 1), lambda b: (b, 0)),
        ],
        out_specs=pl.BlockSpec((G, wo), lambda b: (0, 0)),
        out_shape=jax.ShapeDtypeStruct((G, wo), _f32),
    )(x, batchf)


def _head(pooled, wf0, gb0, wf1, gb1, wout, bout2):
    def body(p_ref, w0_ref, g0_ref, w1_ref, g1_ref, wo_ref, bo_ref, o_ref):
        p = p_ref[...]
        p = jnp.where(jnp.isneginf(p), 0.0, p)
        h = jnp.dot(p, w0_ref[...], preferred_element_type=_f32, precision=lax.Precision.HIGHEST)
        m = jnp.mean(h, 0, keepdims=True)
        v = jnp.mean(h * h, 0, keepdims=True) - m * m
        h = jnp.maximum(
            g0_ref[0:1, :] * (h - m) * lax.rsqrt(v + 1e-5) + g0_ref[1:2, :],
            0.0)
        h2 = jnp.dot(h, w1_ref[...], preferred_element_type=_f32, precision=lax.Precision.HIGHEST)
        m2 = jnp.mean(h2, 0, keepdims=True)
        v2 = jnp.mean(h2 * h2, 0, keepdims=True) - m2 * m2
        h2 = jnp.maximum(
            g1_ref[0:1, :] * (h2 - m2) * lax.rsqrt(v2 + 1e-5) + g1_ref[1:2, :],
            0.0)
        lg = jnp.dot(h2, wo_ref[...], preferred_element_type=_f32, precision=lax.Precision.HIGHEST) + bo_ref[...]
        mx = jnp.max(lg, axis=1, keepdims=True)
        lse = jnp.log(jnp.sum(jnp.exp(lg - mx), axis=1, keepdims=True)) + mx
        o_ref[...] = lg - lse

    return pl.pallas_call(
        body,
        out_shape=jax.ShapeDtypeStruct((G, 10), _f32),
    )(pooled, wf0, gb0, wf1, gb1, wout, bout2)


def _gb(gamma, beta):
    wo = gamma.shape[0]
    return jnp.concatenate(
        [gamma[None, :], beta[None, :], jnp.zeros((6, wo), _f32)], axis=0)


def kernel(x, edge_index, batch,
           Wg0, bg0, gamma0, beta0, Wg1, bg1, gamma1, beta1,
           Wg2, bg2, gamma2, beta2, Wg3, bg3, gamma3, beta3,
           Wf0, bf0, gf0, betf0, Wf1, bf1, gf1, betf1, Wout, bout):
    src = edge_index[0]
    dst = edge_index[1]
    zeros128 = jnp.zeros((RS_BIG, 128), _f32)
    ones128 = jnp.ones((CH, 128), _f32)
    batchf = batch.astype(_f32)[:, None]

    degparts = _sc_degree(dst, ones128, zeros128)
    dinv8, y0 = _finalize_deg(degparts, x)

    (pxr,) = _sc_prop(src, dst, zeros128, [y0])
    px = _combine(pxr, y0, dinv8)

    t0, st0 = _matmul_stats([px], Wg0)
    pool0, ys0 = _bn_apply(t0, st0, _gb(gamma0, beta0), dinv8, batchf, nys=1)
    (p0r,) = _sc_prop(src, dst, zeros128, ys0)
    ph0 = _combine(p0r, ys0[0], dinv8)

    t1, st1 = _matmul_stats([px, ph0], Wg1)
    pool1, ys1 = _bn_apply(t1, st1, _gb(gamma1, beta1), dinv8, batchf, nys=1)
    (p1r,) = _sc_prop(src, dst, zeros128, ys1)
    ph1 = _combine(p1r, ys1[0], dinv8)

    t2, st2 = _matmul_stats([px, ph0, ph1], Wg2)
    pool2, ys2 = _bn_apply(t2, st2, _gb(gamma2, beta2), dinv8, batchf, nys=4)
    p2rs = _sc_prop(src, dst, zeros128, ys2)
    ph2s = [_combine(p2rs[j], ys2[j], dinv8) for j in range(4)]

    t3, st3 = _matmul_stats([px, ph0, ph1] + ph2s, Wg3)
    pool3, _ = _bn_apply(t3, st3, _gb(gamma3, beta3), dinv8, batchf, nys=0)

    poolx = _segmax(x, batchf)
    pooled = jnp.concatenate([poolx, pool0, pool1, pool2, pool3], axis=1)
    return _head(pooled, Wf0, _gb(gf0, betf0), Wf1, _gb(gf1, betf1),
                 Wout, bout[None, :])


# speedup vs baseline: 1.0330x; 1.0330x over previous
import functools

import jax
import jax.numpy as jnp
from jax import lax
from jax.experimental import pallas as pl
from jax.experimental.pallas import tpu as pltpu
from jax.experimental.pallas import tpu_sc as plsc

N = 10000
E = 320000
G = 32
NC = 2
NS = 16
RB = 1000
NB = N // RB
CH = 128
PER_W = E // (NC * NS)
ITERS = 80
NSINK = 8
RS_SMALL = 624
RS_BIG = N - RS_SMALL * (NS - 1)


def _row_split(s, fn):
    @pl.when(s < NS - 1)
    def _():
        fn(s * RS_SMALL, RS_SMALL)

    @pl.when(s == NS - 1)
    def _():
        fn(RS_SMALL * (NS - 1), RS_BIG)

_f32 = jnp.float32


def _sc_mesh():
    return plsc.VectorSubcoreMesh(
        core_axis_name="c", subcore_axis_name="s",
        num_cores=NC, num_subcores=NS)


@functools.lru_cache(maxsize=None)
def _make_degree():
    @functools.partial(
        pl.kernel,
        out_type=jax.ShapeDtypeStruct((NC, N, 128), _f32),
        mesh=_sc_mesh(),
        scratch_types=[
            pltpu.VMEM((ITERS, CH), jnp.int32),
            pltpu.VMEM((CH, 128), _f32),
            pltpu.VMEM_SHARED((N + NSINK, 128), _f32),
            pltpu.SemaphoreType.DMA,
        ],
    )
    def deg_kernel(dstr_hbm, ones_hbm, zeros_hbm, out_hbm,
                   dst_all, ones_v, acc_sh, ssem):
        c = lax.axis_index("c")
        s = lax.axis_index("s")
        wid = c * NS + s
        pltpu.sync_copy(ones_hbm, ones_v)
        pltpu.sync_copy(dstr_hbm.at[wid], dst_all)
        _row_split(s, lambda r0, nr: pltpu.sync_copy(
            zeros_hbm.at[pl.ds(0, nr)], acc_sh.at[pl.ds(r0, nr)]))
        plsc.subcore_barrier()

        def body(i, carry):
            pltpu.async_copy(ones_v, acc_sh.at[dst_all.at[i]],
                             ssem, add=True)
            return carry

        lax.fori_loop(0, ITERS, body, 0)

        def drain(i, carry):
            pltpu.make_async_copy(
                zeros_hbm.at[pl.ds(0, CH)], ones_v, ssem).wait()
            return carry

        lax.fori_loop(0, ITERS, drain, 0)
        plsc.subcore_barrier()
        _row_split(s, lambda r0, nr: pltpu.sync_copy(
            acc_sh.at[pl.ds(r0, nr)], out_hbm.at[c, pl.ds(r0, nr)]))

    return deg_kernel


@functools.lru_cache(maxsize=None)
def _make_prop(num_blocks):
    scratch = [
        pltpu.VMEM((ITERS, CH), jnp.int32),
        pltpu.VMEM((CH,), jnp.int32),
        pltpu.VMEM((CH,), jnp.int32),
        pltpu.VMEM((2, CH, 128), _f32),
        pltpu.VMEM_SHARED((N + NSINK, 128), _f32),
        pltpu.SemaphoreType.DMA,
        pltpu.SemaphoreType.DMA,
    ]
    out_type = [jax.ShapeDtypeStruct((NC, N, 128), _f32)
                for _ in range(num_blocks)]

    @functools.partial(pl.kernel, out_type=out_type, mesh=_sc_mesh(),
                       scratch_types=scratch)
    def prop_kernel(srcr_hbm, dstr_hbm, zeros_hbm, *rest):
        pieces = rest[:num_blocks]
        outs = rest[num_blocks:2 * num_blocks]
        src_all, dstb0, dstb1, rows, acc_sh, gsem, isem = \
            rest[2 * num_blocks:]
        dstb = (dstb0, dstb1)
        c = lax.axis_index("c")
        s = lax.axis_index("s")
        wid = c * NS + s
        pltpu.sync_copy(srcr_hbm.at[wid], src_all)

        def wait_idx():
            pltpu.make_async_copy(dstr_hbm.at[0], dstb0, isem).wait()

        def wait_gather():
            pltpu.make_async_copy(
                zeros_hbm.at[pl.ds(0, CH)], rows.at[0], gsem).wait()

        for b in range(num_blocks):
            _row_split(s, lambda r0, nr: pltpu.sync_copy(
                zeros_hbm.at[pl.ds(0, nr)], acc_sh.at[pl.ds(r0, nr)]))
            pltpu.async_copy(dstr_hbm.at[wid * ITERS], dstb0, isem)
            pltpu.async_copy(pieces[b].at[src_all.at[0]], rows.at[0], gsem)
            plsc.subcore_barrier()

            def chunk(i, p, b):
                wait_idx()
                wait_gather()

                @pl.when(i + 1 < ITERS)
                def _():
                    pltpu.async_copy(dstr_hbm.at[wid * ITERS + i + 1],
                                     dstb[1 - p], isem)
                    pltpu.async_copy(pieces[b].at[src_all.at[i + 1]],
                                     rows.at[1 - p], gsem)

                pltpu.sync_copy(rows.at[p], acc_sh.at[dstb[p]], add=True)

            def body(k, carry, b=b):
                chunk(2 * k, 0, b)
                chunk(2 * k + 1, 1, b)
                return carry

            lax.fori_loop(0, ITERS // 2, body, 0)
            plsc.subcore_barrier()
            _row_split(s, lambda r0, nr, b=b: pltpu.sync_copy(
                acc_sh.at[pl.ds(r0, nr)], outs[b].at[c, pl.ds(r0, nr)]))
            plsc.subcore_barrier()

    return prop_kernel


def _sc_degree(dst, ones128, zeros128):
    dstr = _pad_idx(dst, N)
    return _make_degree()(dstr, ones128, zeros128)


def _pad_idx(idx, fill):
    per_w = idx.reshape(NC * NS, PER_W)
    padded = jnp.pad(per_w, ((0, 0), (0, ITERS * CH - PER_W)),
                     constant_values=fill)
    return padded.reshape(NC * NS, ITERS, CH)


def _sc_prop(src, dst, zeros128, pieces):
    srcr = _pad_idx(src, 0)
    dstr = _pad_idx(dst, N).reshape(-1, CH)
    outs = _make_prop(len(pieces))(srcr, dstr, zeros128, *pieces)
    return list(outs)


def _finalize_deg(degparts, x):
    def body(dp_ref, x_ref, dinv_ref, y0_ref):
        deg = dp_ref[0, :, 0:1] + dp_ref[1, :, 0:1] + 1.0
        dinv = lax.rsqrt(deg)
        dinv_ref[...] = jnp.broadcast_to(dinv, (RB, 8))
        y0_ref[...] = x_ref[...] * dinv

    return pl.pallas_call(
        body,
        grid=(NB,),
        in_specs=[
            pl.BlockSpec((NC, RB, 128), lambda b: (0, b, 0)),
            pl.BlockSpec((RB, 128), lambda b: (b, 0)),
        ],
        out_specs=[
            pl.BlockSpec((RB, 8), lambda b: (b, 0)),
            pl.BlockSpec((RB, 128), lambda b: (b, 0)),
        ],
        out_shape=[
            jax.ShapeDtypeStruct((N, 8), _f32),
            jax.ShapeDtypeStruct((N, 128), _f32),
        ],
    )(degparts, x)


def _combine(parts, y, dinv8):
    def body(p_ref, y_ref, dinv_ref, o_ref):
        o_ref[...] = ((p_ref[0] + p_ref[1] + y_ref[...])
                      * dinv_ref[:, 0:1])

    return pl.pallas_call(
        body,
        grid=(NB,),
        in_specs=[
            pl.BlockSpec((NC, RB, 128), lambda b: (0, b, 0)),
            pl.BlockSpec((RB, 128), lambda b: (b, 0)),
            pl.BlockSpec((RB, 8), lambda b: (b, 0)),
        ],
        out_specs=pl.BlockSpec((RB, 128), lambda b: (b, 0)),
        out_shape=jax.ShapeDtypeStruct((N, 128), _f32),
    )(parts, y, dinv8)


def _matmul_stats(pieces, w):
    npc = len(pieces)
    wo = w.shape[1]

    def body(*refs):
        piece_refs = refs[:npc]
        w_ref = refs[npc]
        t_ref, st_ref = refs[npc + 1:]
        b = pl.program_id(0)
        acc = jnp.zeros((RB, wo), _f32)
        for j in range(npc):
            acc = acc + jnp.dot(piece_refs[j][...],
                                w_ref[j * 128:(j + 1) * 128, :],
                                preferred_element_type=_f32,
                                precision=lax.Precision.HIGHEST)
        t_ref[...] = acc

        @pl.when(b == 0)
        def _():
            st_ref[...] = jnp.zeros((8, wo), _f32)

        st_ref[0:1, :] = st_ref[0:1, :] + jnp.sum(acc, 0, keepdims=True)
        st_ref[1:2, :] = st_ref[1:2, :] + jnp.sum(acc * acc, 0, keepdims=True)

    return pl.pallas_call(
        body,
        grid=(NB,),
        in_specs=[pl.BlockSpec((RB, 128), lambda b: (b, 0))
                  for _ in range(npc)]
                 + [pl.BlockSpec((128 * npc, wo), lambda b: (0, 0))],
        out_specs=[
            pl.BlockSpec((RB, wo), lambda b: (b, 0)),
            pl.BlockSpec((8, wo), lambda b: (0, 0)),
        ],
        out_shape=[
            jax.ShapeDtypeStruct((N, wo), _f32),
            jax.ShapeDtypeStruct((8, wo), _f32),
        ],
    )(*pieces, w)


def _bn_apply(t, stats, gb, dinv8, batchf, nys):
    wo = t.shape[1]

    def body(*refs):
        t_ref, st_ref, gb_ref, dinv_ref, bf_ref, pool_ref = refs[:6]
        ys_refs = refs[6:]
        b = pl.program_id(0)
        m = st_ref[0:1, :] * (1.0 / N)
        var = st_ref[1:2, :] * (1.0 / N) - m * m
        t_blk = t_ref[...]
        h = gb_ref[0:1, :] * (t_blk - m) * lax.rsqrt(var + 1e-5) + gb_ref[1:2, :]
        h = jnp.maximum(h, 0.0)
        for j in range(nys):
            ys_refs[j][...] = h[:, j * 128:(j + 1) * 128] * dinv_ref[:, 0:1]

        @pl.when(b == 0)
        def _():
            pool_ref[...] = jnp.full((G, wo), -jnp.inf, _f32)

        bf = bf_ref[...]
        for g in range(G):
            hm = jnp.where(bf == float(g), h, -jnp.inf)
            mx = jnp.max(hm, axis=0, keepdims=True)
            pool_ref[g:g + 1, :] = jnp.maximum(pool_ref[g:g + 1, :], mx)

    outs = pl.pallas_call(
        body,
        grid=(NB,),
        in_specs=[
            pl.BlockSpec((RB, wo), lambda b: (b, 0)),
            pl.BlockSpec((8, wo), lambda b: (0, 0)),
            pl.BlockSpec((8, wo), lambda b: (0, 0)),
            pl.BlockSpec((RB, 8), lambda b: (b, 0)),
            pl.BlockSpec((RB, 1), lambda b: (b, 0)),
        ],
        out_specs=[pl.BlockSpec((G, wo), lambda b: (0, 0))]
                  + [pl.BlockSpec((RB, 128), lambda b: (b, 0))
                     for _ in range(nys)],
        out_shape=[jax.ShapeDtypeStruct((G, wo), _f32)]
                  + [jax.ShapeDtypeStruct((N, 128), _f32)
                     for _ in range(nys)],
    )(t, stats, gb, dinv8, batchf)
    return outs[0], list(outs[1:])


def _segmax(x, batchf):
    wo = x.shape[1]

    def body(x_ref, bf_ref, pool_ref):
        b = pl.program_id(0)

        @pl.when(b == 0)
        def _():
            pool_ref[...] = jnp.full((G, wo), -jnp.inf, _f32)

        h = x_ref[...]
        bf = bf_ref[...]
        for g in range(G):
            hm = jnp.where(bf == float(g), h, -jnp.inf)
            mx = jnp.max(hm, axis=0, keepdims=True)
            pool_ref[g:g + 1, :] = jnp.maximum(pool_ref[g:g + 1, :], mx)

    return pl.pallas_call(
        body,
        grid=(NB,),
        in_specs=[
            pl.BlockSpec((RB, wo), lambda b: (b, 0)),
            pl.BlockSpec((RB, 1), lambda b: (b, 0)),
        ],
        out_specs=pl.BlockSpec((G, wo), lambda b: (0, 0)),
        out_shape=jax.ShapeDtypeStruct((G, wo), _f32),
    )(x, batchf)


def _head(pooled, wf0, gb0, wf1, gb1, wout, bout2):
    def body(p_ref, w0_ref, g0_ref, w1_ref, g1_ref, wo_ref, bo_ref, o_ref):
        p = p_ref[...]
        p = jnp.where(jnp.isneginf(p), 0.0, p)
        h = jnp.dot(p, w0_ref[...], preferred_element_type=_f32, precision=lax.Precision.HIGHEST)
        m = jnp.mean(h, 0, keepdims=True)
        v = jnp.mean(h * h, 0, keepdims=True) - m * m
        h = jnp.maximum(
            g0_ref[0:1, :] * (h - m) * lax.rsqrt(v + 1e-5) + g0_ref[1:2, :],
            0.0)
        h2 = jnp.dot(h, w1_ref[...], preferred_element_type=_f32, precision=lax.Precision.HIGHEST)
        m2 = jnp.mean(h2, 0, keepdims=True)
        v2 = jnp.mean(h2 * h2, 0, keepdims=True) - m2 * m2
        h2 = jnp.maximum(
            g1_ref[0:1, :] * (h2 - m2) * lax.rsqrt(v2 + 1e-5) + g1_ref[1:2, :],
            0.0)
        lg = jnp.dot(h2, wo_ref[...], preferred_element_type=_f32, precision=lax.Precision.HIGHEST) + bo_ref[...]
        mx = jnp.max(lg, axis=1, keepdims=True)
        lse = jnp.log(jnp.sum(jnp.exp(lg - mx), axis=1, keepdims=True)) + mx
        o_ref[...] = lg - lse

    return pl.pallas_call(
        body,
        out_shape=jax.ShapeDtypeStruct((G, 10), _f32),
    )(pooled, wf0, gb0, wf1, gb1, wout, bout2)


def _gb(gamma, beta):
    wo = gamma.shape[0]
    return jnp.concatenate(
        [gamma[None, :], beta[None, :], jnp.zeros((6, wo), _f32)], axis=0)


def kernel(x, edge_index, batch,
           Wg0, bg0, gamma0, beta0, Wg1, bg1, gamma1, beta1,
           Wg2, bg2, gamma2, beta2, Wg3, bg3, gamma3, beta3,
           Wf0, bf0, gf0, betf0, Wf1, bf1, gf1, betf1, Wout, bout):
    src = edge_index[0]
    dst = edge_index[1]
    zeros128 = jnp.zeros((RS_BIG, 128), _f32)
    ones128 = jnp.ones((CH, 128), _f32)
    batchf = batch.astype(_f32)[:, None]

    degparts = _sc_degree(dst, ones128, zeros128)
    dinv8, y0 = _finalize_deg(degparts, x)

    (pxr,) = _sc_prop(src, dst, zeros128, [y0])
    px = _combine(pxr, y0, dinv8)

    t0, st0 = _matmul_stats([px], Wg0)
    pool0, ys0 = _bn_apply(t0, st0, _gb(gamma0, beta0), dinv8, batchf, nys=1)
    (p0r,) = _sc_prop(src, dst, zeros128, ys0)
    ph0 = _combine(p0r, ys0[0], dinv8)

    t1, st1 = _matmul_stats([px, ph0], Wg1)
    pool1, ys1 = _bn_apply(t1, st1, _gb(gamma1, beta1), dinv8, batchf, nys=1)
    (p1r,) = _sc_prop(src, dst, zeros128, ys1)
    ph1 = _combine(p1r, ys1[0], dinv8)

    t2, st2 = _matmul_stats([px, ph0, ph1], Wg2)
    pool2, ys2 = _bn_apply(t2, st2, _gb(gamma2, beta2), dinv8, batchf, nys=4)
    p2rs = _sc_prop(src, dst, zeros128, ys2)
    ph2s = [_combine(p2rs[j], ys2[j], dinv8) for j in range(4)]

    t3, st3 = _matmul_stats([px, ph0, ph1] + ph2s, Wg3)
    pool3, _ = _bn_apply(t3, st3, _gb(gamma3, beta3), dinv8, batchf, nys=0)

    poolx = _segmax(x, batchf)
    pooled = jnp.concatenate([poolx, pool0, pool1, pool2, pool3], axis=1)
    return _head(pooled, Wf0, _gb(gf0, betf0), Wf1, _gb(gf1, betf1),
                 Wout, bout[None, :])


# speedup vs baseline: 1.2793x; 1.2384x over previous
import functools

import jax
import jax.numpy as jnp
from jax import lax
from jax.experimental import pallas as pl
from jax.experimental.pallas import tpu as pltpu
from jax.experimental.pallas import tpu_sc as plsc

N = 10000
E = 320000
G = 32
NC = 2
NS = 16
RB = 1000
NB = N // RB
CH = 128
PER_W = E // (NC * NS)
ITERS = 80
NSINK = 8
CHP = 80
ITERSP = PER_W // CHP
RS_SMALL = 624
RS_BIG = N - RS_SMALL * (NS - 1)


def _row_split(s, fn):
    @pl.when(s < NS - 1)
    def _():
        fn(s * RS_SMALL, RS_SMALL)

    @pl.when(s == NS - 1)
    def _():
        fn(RS_SMALL * (NS - 1), RS_BIG)

_f32 = jnp.float32


def _sc_mesh():
    return plsc.VectorSubcoreMesh(
        core_axis_name="c", subcore_axis_name="s",
        num_cores=NC, num_subcores=NS)


@functools.lru_cache(maxsize=None)
def _make_degree():
    @functools.partial(
        pl.kernel,
        out_type=jax.ShapeDtypeStruct((NC, N, 128), _f32),
        mesh=_sc_mesh(),
        scratch_types=[
            pltpu.VMEM((ITERS, CH), jnp.int32),
            pltpu.VMEM((CH, 128), _f32),
            pltpu.VMEM_SHARED((N + NSINK, 128), _f32),
            pltpu.SemaphoreType.DMA,
        ],
    )
    def deg_kernel(dstr_hbm, ones_hbm, zeros_hbm, out_hbm,
                   dst_all, ones_v, acc_sh, ssem):
        c = lax.axis_index("c")
        s = lax.axis_index("s")
        wid = c * NS + s
        pltpu.sync_copy(ones_hbm, ones_v)
        pltpu.sync_copy(dstr_hbm.at[wid], dst_all)
        _row_split(s, lambda r0, nr: pltpu.sync_copy(
            zeros_hbm.at[pl.ds(0, nr)], acc_sh.at[pl.ds(r0, nr)]))
        plsc.subcore_barrier()

        def body(i, carry):
            pltpu.async_copy(ones_v, acc_sh.at[dst_all.at[i]],
                             ssem, add=True)
            return carry

        lax.fori_loop(0, ITERS, body, 0)

        def drain(i, carry):
            pltpu.make_async_copy(
                zeros_hbm.at[pl.ds(0, CH)], ones_v, ssem).wait()
            return carry

        lax.fori_loop(0, ITERS, drain, 0)
        plsc.subcore_barrier()
        _row_split(s, lambda r0, nr: pltpu.sync_copy(
            acc_sh.at[pl.ds(r0, nr)], out_hbm.at[c, pl.ds(r0, nr)]))

    return deg_kernel


@functools.lru_cache(maxsize=None)
def _make_prop(num_blocks):
    scratch = [
        pltpu.VMEM((CHP,), jnp.int32),
        pltpu.VMEM((CHP,), jnp.int32),
        pltpu.VMEM((CHP, 128), _f32),
        pltpu.VMEM_SHARED((N, 128), _f32),
        pltpu.SemaphoreType.DMA,
    ]
    out_type = [jax.ShapeDtypeStruct((NC, N, 128), _f32)
                for _ in range(num_blocks)]

    @functools.partial(pl.kernel, out_type=out_type, mesh=_sc_mesh(),
                       scratch_types=scratch)
    def prop_kernel(src_hbm, dst_hbm, zeros_hbm, *rest):
        pieces = rest[:num_blocks]
        outs = rest[num_blocks:2 * num_blocks]
        src_v, dst_v, rows_v, acc_sh, sem = rest[2 * num_blocks:]
        c = lax.axis_index("c")
        s = lax.axis_index("s")
        ebase = (c * NS + s) * PER_W
        for b in range(num_blocks):
            _row_split(s, lambda r0, nr: pltpu.sync_copy(
                zeros_hbm.at[pl.ds(0, nr)], acc_sh.at[pl.ds(r0, nr)]))
            plsc.subcore_barrier()

            def body(i, carry, b=b):
                base = ebase + i * CHP
                pltpu.sync_copy(src_hbm.at[pl.ds(base, CHP)], src_v)
                pltpu.sync_copy(dst_hbm.at[pl.ds(base, CHP)], dst_v)
                pltpu.async_copy(pieces[b].at[src_v], rows_v, sem).wait()
                pltpu.sync_copy(rows_v, acc_sh.at[dst_v], add=True)
                return carry

            lax.fori_loop(0, ITERSP, body, 0)
            plsc.subcore_barrier()
            _row_split(s, lambda r0, nr, b=b: pltpu.sync_copy(
                acc_sh.at[pl.ds(r0, nr)], outs[b].at[c, pl.ds(r0, nr)]))
            plsc.subcore_barrier()

    return prop_kernel


def _sc_degree(dst, ones128, zeros128):
    dstr = _pad_idx(dst, N)
    return _make_degree()(dstr, ones128, zeros128)


def _pad_idx(idx, fill):
    per_w = idx.reshape(NC * NS, PER_W)
    padded = jnp.pad(per_w, ((0, 0), (0, ITERS * CH - PER_W)),
                     constant_values=fill)
    return padded.reshape(NC * NS, ITERS, CH)


def _sc_prop(src, dst, zeros128, pieces):
    outs = _make_prop(len(pieces))(src, dst, zeros128, *pieces)
    return list(outs)


def _finalize_deg(degparts, x):
    def body(dp_ref, x_ref, dinv_ref, y0_ref):
        deg = dp_ref[0, :, 0:1] + dp_ref[1, :, 0:1] + 1.0
        dinv = lax.rsqrt(deg)
        dinv_ref[...] = jnp.broadcast_to(dinv, (RB, 8))
        y0_ref[...] = x_ref[...] * dinv

    return pl.pallas_call(
        body,
        grid=(NB,),
        in_specs=[
            pl.BlockSpec((NC, RB, 128), lambda b: (0, b, 0)),
            pl.BlockSpec((RB, 128), lambda b: (b, 0)),
        ],
        out_specs=[
            pl.BlockSpec((RB, 8), lambda b: (b, 0)),
            pl.BlockSpec((RB, 128), lambda b: (b, 0)),
        ],
        out_shape=[
            jax.ShapeDtypeStruct((N, 8), _f32),
            jax.ShapeDtypeStruct((N, 128), _f32),
        ],
    )(degparts, x)


def _combine(parts, y, dinv8):
    def body(p_ref, y_ref, dinv_ref, o_ref):
        o_ref[...] = ((p_ref[0] + p_ref[1] + y_ref[...])
                      * dinv_ref[:, 0:1])

    return pl.pallas_call(
        body,
        grid=(NB,),
        in_specs=[
            pl.BlockSpec((NC, RB, 128), lambda b: (0, b, 0)),
            pl.BlockSpec((RB, 128), lambda b: (b, 0)),
            pl.BlockSpec((RB, 8), lambda b: (b, 0)),
        ],
        out_specs=pl.BlockSpec((RB, 128), lambda b: (b, 0)),
        out_shape=jax.ShapeDtypeStruct((N, 128), _f32),
    )(parts, y, dinv8)


def _matmul_stats(pieces, w):
    npc = len(pieces)
    wo = w.shape[1]

    def body(*refs):
        piece_refs = refs[:npc]
        w_ref = refs[npc]
        t_ref, st_ref = refs[npc + 1:]
        b = pl.program_id(0)
        acc = jnp.zeros((RB, wo), _f32)
        for j in range(npc):
            acc = acc + jnp.dot(piece_refs[j][...],
                                w_ref[j * 128:(j + 1) * 128, :],
                                preferred_element_type=_f32,
                                precision=lax.Precision.HIGHEST)
        t_ref[...] = acc

        @pl.when(b == 0)
        def _():
            st_ref[...] = jnp.zeros((8, wo), _f32)

        st_ref[0:1, :] = st_ref[0:1, :] + jnp.sum(acc, 0, keepdims=True)
        st_ref[1:2, :] = st_ref[1:2, :] + jnp.sum(acc * acc, 0, keepdims=True)

    return pl.pallas_call(
        body,
        grid=(NB,),
        in_specs=[pl.BlockSpec((RB, 128), lambda b: (b, 0))
                  for _ in range(npc)]
                 + [pl.BlockSpec((128 * npc, wo), lambda b: (0, 0))],
        out_specs=[
            pl.BlockSpec((RB, wo), lambda b: (b, 0)),
            pl.BlockSpec((8, wo), lambda b: (0, 0)),
        ],
        out_shape=[
            jax.ShapeDtypeStruct((N, wo), _f32),
            jax.ShapeDtypeStruct((8, wo), _f32),
        ],
    )(*pieces, w)


def _bn_apply(t, stats, gb, dinv8, batchf, nys):
    wo = t.shape[1]

    def body(*refs):
        t_ref, st_ref, gb_ref, dinv_ref, bf_ref, pool_ref = refs[:6]
        ys_refs = refs[6:]
        b = pl.program_id(0)
        m = st_ref[0:1, :] * (1.0 / N)
        var = st_ref[1:2, :] * (1.0 / N) - m * m
        t_blk = t_ref[...]
        h = gb_ref[0:1, :] * (t_blk - m) * lax.rsqrt(var + 1e-5) + gb_ref[1:2, :]
        h = jnp.maximum(h, 0.0)
        for j in range(nys):
            ys_refs[j][...] = h[:, j * 128:(j + 1) * 128] * dinv_ref[:, 0:1]

        @pl.when(b == 0)
        def _():
            pool_ref[...] = jnp.full((G, wo), -jnp.inf, _f32)

        bf = bf_ref[...]
        for g in range(G):
            hm = jnp.where(bf == float(g), h, -jnp.inf)
            mx = jnp.max(hm, axis=0, keepdims=True)
            pool_ref[g:g + 1, :] = jnp.maximum(pool_ref[g:g + 1, :], mx)

    outs = pl.pallas_call(
        body,
        grid=(NB,),
        in_specs=[
            pl.BlockSpec((RB, wo), lambda b: (b, 0)),
            pl.BlockSpec((8, wo), lambda b: (0, 0)),
            pl.BlockSpec((8, wo), lambda b: (0, 0)),
            pl.BlockSpec((RB, 8), lambda b: (b, 0)),
            pl.BlockSpec((RB, 1), lambda b: (b, 0)),
        ],
        out_specs=[pl.BlockSpec((G, wo), lambda b: (0, 0))]
                  + [pl.BlockSpec((RB, 128), lambda b: (b, 0))
                     for _ in range(nys)],
        out_shape=[jax.ShapeDtypeStruct((G, wo), _f32)]
                  + [jax.ShapeDtypeStruct((N, 128), _f32)
                     for _ in range(nys)],
    )(t, stats, gb, dinv8, batchf)
    return outs[0], list(outs[1:])


def _segmax(x, batchf):
    wo = x.shape[1]

    def body(x_ref, bf_ref, pool_ref):
        b = pl.program_id(0)

        @pl.when(b == 0)
        def _():
            pool_ref[...] = jnp.full((G, wo), -jnp.inf, _f32)

        h = x_ref[...]
        bf = bf_ref[...]
        for g in range(G):
            hm = jnp.where(bf == float(g), h, -jnp.inf)
            mx = jnp.max(hm, axis=0, keepdims=True)
            pool_ref[g:g + 1, :] = jnp.maximum(pool_ref[g:g + 1, :], mx)

    return pl.pallas_call(
        body,
        grid=(NB,),
        in_specs=[
            pl.BlockSpec((RB, wo), lambda b: (b, 0)),
            pl.BlockSpec((RB, 1), lambda b: (b, 0)),
        ],
        out_specs=pl.BlockSpec((G, wo), lambda b: (0, 0)),
        out_shape=jax.ShapeDtypeStruct((G, wo), _f32),
    )(x, batchf)


def _head(pooled, wf0, gb0, wf1, gb1, wout, bout2):
    def body(p_ref, w0_ref, g0_ref, w1_ref, g1_ref, wo_ref, bo_ref, o_ref):
        p = p_ref[...]
        p = jnp.where(jnp.isneginf(p), 0.0, p)
        h = jnp.dot(p, w0_ref[...], preferred_element_type=_f32, precision=lax.Precision.HIGHEST)
        m = jnp.mean(h, 0, keepdims=True)
        v = jnp.mean(h * h, 0, keepdims=True) - m * m
        h = jnp.maximum(
            g0_ref[0:1, :] * (h - m) * lax.rsqrt(v + 1e-5) + g0_ref[1:2, :],
            0.0)
        h2 = jnp.dot(h, w1_ref[...], preferred_element_type=_f32, precision=lax.Precision.HIGHEST)
        m2 = jnp.mean(h2, 0, keepdims=True)
        v2 = jnp.mean(h2 * h2, 0, keepdims=True) - m2 * m2
        h2 = jnp.maximum(
            g1_ref[0:1, :] * (h2 - m2) * lax.rsqrt(v2 + 1e-5) + g1_ref[1:2, :],
            0.0)
        lg = jnp.dot(h2, wo_ref[...], preferred_element_type=_f32, precision=lax.Precision.HIGHEST) + bo_ref[...]
        mx = jnp.max(lg, axis=1, keepdims=True)
        lse = jnp.log(jnp.sum(jnp.exp(lg - mx), axis=1, keepdims=True)) + mx
        o_ref[...] = lg - lse

    return pl.pallas_call(
        body,
        out_shape=jax.ShapeDtypeStruct((G, 10), _f32),
    )(pooled, wf0, gb0, wf1, gb1, wout, bout2)


def _gb(gamma, beta):
    wo = gamma.shape[0]
    return jnp.concatenate(
        [gamma[None, :], beta[None, :], jnp.zeros((6, wo), _f32)], axis=0)


def kernel(x, edge_index, batch,
           Wg0, bg0, gamma0, beta0, Wg1, bg1, gamma1, beta1,
           Wg2, bg2, gamma2, beta2, Wg3, bg3, gamma3, beta3,
           Wf0, bf0, gf0, betf0, Wf1, bf1, gf1, betf1, Wout, bout):
    src = edge_index[0]
    dst = edge_index[1]
    zeros128 = jnp.zeros((RS_BIG, 128), _f32)
    ones128 = jnp.ones((CH, 128), _f32)
    batchf = batch.astype(_f32)[:, None]

    degparts = _sc_degree(dst, ones128, zeros128)
    dinv8, y0 = _finalize_deg(degparts, x)

    (pxr,) = _sc_prop(src, dst, zeros128, [y0])
    px = _combine(pxr, y0, dinv8)

    t0, st0 = _matmul_stats([px], Wg0)
    pool0, ys0 = _bn_apply(t0, st0, _gb(gamma0, beta0), dinv8, batchf, nys=1)
    (p0r,) = _sc_prop(src, dst, zeros128, ys0)
    ph0 = _combine(p0r, ys0[0], dinv8)

    t1, st1 = _matmul_stats([px, ph0], Wg1)
    pool1, ys1 = _bn_apply(t1, st1, _gb(gamma1, beta1), dinv8, batchf, nys=1)
    (p1r,) = _sc_prop(src, dst, zeros128, ys1)
    ph1 = _combine(p1r, ys1[0], dinv8)

    t2, st2 = _matmul_stats([px, ph0, ph1], Wg2)
    pool2, ys2 = _bn_apply(t2, st2, _gb(gamma2, beta2), dinv8, batchf, nys=4)
    p2rs = _sc_prop(src, dst, zeros128, ys2)
    ph2s = [_combine(p2rs[j], ys2[j], dinv8) for j in range(4)]

    t3, st3 = _matmul_stats([px, ph0, ph1] + ph2s, Wg3)
    pool3, _ = _bn_apply(t3, st3, _gb(gamma3, beta3), dinv8, batchf, nys=0)

    poolx = _segmax(x, batchf)
    pooled = jnp.concatenate([poolx, pool0, pool1, pool2, pool3], axis=1)
    return _head(pooled, Wf0, _gb(gf0, betf0), Wf1, _gb(gf1, betf1),
                 Wout, bout[None, :])
